# baseline (speedup 1.0000x reference)
import functools

import jax
import jax.numpy as jnp
from jax.experimental import pallas as pl
from jax.experimental.pallas import tpu as pltpu

_NUM_PERS = 4
_D = 64
_TOPK = 16
_NEG = -3.0


def _normalize_feats(x, w, scale=1.0):
    sq = jax.lax.dot_general(
        x * x, w * w,
        dimension_numbers=(((1,), (1,)), ((), ())),
        precision=jax.lax.Precision.HIGHEST,
        preferred_element_type=jnp.float32,
    )
    inv = scale / jnp.maximum(jnp.sqrt(sq), 1e-12)
    feats = []
    for p in range(_NUM_PERS):
        feats.append((x * w[p][None, :] * inv[:, p:p + 1]).astype(jnp.bfloat16))
    return jnp.concatenate(feats, axis=1)


def _block_kernel(nodes_ref, pivots_ref, w_ref, out_ref, pfeat_ref):
    pid = pl.program_id(0)

    @pl.when(pid == 0)
    def _():
        pfeat_ref[...] = _normalize_feats(pivots_ref[...], w_ref[...], scale=0.25)

    nfeat = _normalize_feats(nodes_ref[...], w_ref[...])
    scores = jax.lax.dot_general(
        nfeat, pfeat_ref[...],
        dimension_numbers=(((1,), (1,)), ((), ())),
        preferred_element_type=jnp.float32,
    )

    b = scores
    for _ in range(_TOPK):
        m = jnp.max(b, axis=1, keepdims=True)
        b = jnp.where(b == m, _NEG, b)
    out_ref[...] = jnp.where(b == _NEG, scores, 0.0)


@jax.jit
def kernel(nodes, pivots, weight_tensor):
    n, d = nodes.shape
    m = pivots.shape[0]
    bn = 400
    grid = n // bn
    return pl.pallas_call(
        _block_kernel,
        grid=(grid,),
        in_specs=[
            pl.BlockSpec((bn, d), lambda i: (i, 0)),
            pl.BlockSpec((m, d), lambda i: (0, 0)),
            pl.BlockSpec((_NUM_PERS, d), lambda i: (0, 0)),
        ],
        out_specs=pl.BlockSpec((bn, m), lambda i: (i, 0)),
        out_shape=jax.ShapeDtypeStruct((n, m), jnp.float32),
        scratch_shapes=[pltpu.VMEM((m, _NUM_PERS * d), jnp.bfloat16)],
    )(nodes, pivots, weight_tensor)
